# baseline (speedup 1.0000x reference)
import functools

import jax
import jax.numpy as jnp
from jax.experimental import pallas as pl
from jax.experimental.pallas import tpu as pltpu

N_NODES = 100000
N_EDGES = 1600000
NP = 100352
EB = 8000
NB = 2048
F = 16


def _pad2(a, rows, cols):
    r, c = a.shape
    return jnp.pad(a, ((0, rows - r), (0, cols - c)))


def _pool_weights(wx, bx, we, be, pool):
    ox, ix = wx.shape
    oe, ie = we.shape
    C = ox + oe
    wcat_x = jnp.concatenate([wx.T, jnp.zeros((ix, oe), wx.dtype)], axis=1)
    wcat_e = jnp.concatenate([jnp.zeros((ie, ox), we.dtype), we.T], axis=1)
    bcat = jnp.concatenate([bx, be])
    As, Bs, cs = [], [], []
    for k in range(3):
        if pool:
            cols = [3 * t + k for t in range(C // 3)]
            Ak = wcat_x[:, jnp.array(cols)]
            Bk = wcat_e[:, jnp.array(cols)]
            ck = bcat[jnp.array(cols)]
        else:
            Ak, Bk, ck = wcat_x, wcat_e, bcat
        As.append(_pad2(Ak, F, F))
        Bs.append(_pad2(Bk, F, F))
        cs.append(jnp.pad(ck, (0, F - ck.shape[0])))
    return (jnp.stack(As), jnp.stack(Bs), jnp.stack(cs))


def _pair_body(s0_ref, s1_ref, ea_ref, A_ref, B_ref, c_ref, o_ref):
    s = s0_ref[...] + s1_ref[...]
    ea = ea_ref[...]
    ec = ea.shape[1]
    z = None
    for k in range(3):
        zk = (jnp.dot(s, A_ref[k], preferred_element_type=jnp.float32)
              + jnp.dot(ea, B_ref[k][:ec, :], preferred_element_type=jnp.float32)
              + c_ref[k][None, :])
        z = zk if z is None else jnp.maximum(z, zk)
    o_ref[...] = jnp.maximum(z, 0.0)


def _single_body(s_ref, ea_ref, A_ref, B_ref, c_ref, o_ref):
    s = s_ref[...]
    ea = ea_ref[...]
    ec = ea.shape[1]
    z = None
    for k in range(3):
        zk = (jnp.dot(s, A_ref[k], preferred_element_type=jnp.float32)
              + jnp.dot(ea, B_ref[k][:ec, :], preferred_element_type=jnp.float32)
              + c_ref[k][None, :])
        z = zk if z is None else jnp.maximum(z, zk)
    o_ref[...] = jnp.maximum(z, 0.0)


def _fused_pair(s0, s1, ea, W, rows, blk):
    A, B, c = W
    nb = rows // blk
    eac = ea.shape[1]
    return pl.pallas_call(
        _pair_body,
        grid=(nb,),
        in_specs=[
            pl.BlockSpec((blk, F), lambda i: (i, 0)),
            pl.BlockSpec((blk, F), lambda i: (i, 0)),
            pl.BlockSpec((blk, eac), lambda i: (i, 0)),
            pl.BlockSpec((3, F, F), lambda i: (0, 0, 0)),
            pl.BlockSpec((3, F, F), lambda i: (0, 0, 0)),
            pl.BlockSpec((3, F), lambda i: (0, 0)),
        ],
        out_specs=pl.BlockSpec((blk, F), lambda i: (i, 0)),
        out_shape=jax.ShapeDtypeStruct((rows, F), jnp.float32),
    )(s0, s1, ea, A, B, c)


def _fused_single(s, ea, W, rows, blk):
    A, B, c = W
    nb = rows // blk
    eac = ea.shape[1]
    return pl.pallas_call(
        _single_body,
        grid=(nb,),
        in_specs=[
            pl.BlockSpec((blk, F), lambda i: (i, 0)),
            pl.BlockSpec((blk, eac), lambda i: (i, 0)),
            pl.BlockSpec((3, F, F), lambda i: (0, 0, 0)),
            pl.BlockSpec((3, F, F), lambda i: (0, 0, 0)),
            pl.BlockSpec((3, F), lambda i: (0, 0)),
        ],
        out_specs=pl.BlockSpec((blk, F), lambda i: (i, 0)),
        out_shape=jax.ShapeDtypeStruct((rows, F), jnp.float32),
    )(s, ea, A, B, c)


def _mlp_body(a_ref, w1_ref, b1_ref, w2_ref, b2_ref, w3_ref, b3_ref,
              w4_ref, b4_ref, w5_ref, b5_ref, o_ref):
    h = jnp.maximum(jnp.dot(a_ref[...], w1_ref[...],
                            preferred_element_type=jnp.float32) + b1_ref[...], 0.0)
    h = jnp.maximum(jnp.dot(h, w2_ref[...],
                            preferred_element_type=jnp.float32) + b2_ref[...], 0.0)
    h = jnp.maximum(jnp.dot(h, w3_ref[...],
                            preferred_element_type=jnp.float32) + b3_ref[...], 0.0)
    h = jnp.maximum(jnp.dot(h, w4_ref[...],
                            preferred_element_type=jnp.float32) + b4_ref[...], 0.0)
    h = jnp.dot(h, w5_ref[...], preferred_element_type=jnp.float32) + b5_ref[...]
    o_ref[...] = jnp.where(h > 0, h, 0.2 * h)


def _mlp(act, ws):
    (w1, b1, w2, b2, w3, b3, w4, b4, w5, b5) = ws
    nb = NP // NB
    specs = [pl.BlockSpec((NB, F), lambda i: (i, 0))]
    for w, b in ((w1, b1), (w2, b2), (w3, b3), (w4, b4), (w5, b5)):
        specs.append(pl.BlockSpec(w.shape, lambda i: (0, 0)))
        specs.append(pl.BlockSpec(b.shape, lambda i: (0, 0)))
    return pl.pallas_call(
        _mlp_body,
        grid=(nb,),
        in_specs=specs,
        out_specs=pl.BlockSpec((NB, 8), lambda i: (i, 0)),
        out_shape=jax.ShapeDtypeStruct((NP, 8), jnp.float32),
    )(act, w1, b1, w2, b2, w3, b3, w4, b4, w5, b5)


def kernel(x, edge_index, edge_attr, params):
    ei = edge_index.astype(jnp.int32)
    src, dst = ei[0], ei[1]

    enc = {}
    for name, pool in (("edge1", True), ("node1", True), ("edge2", True),
                       ("node2", True), ("edge3", True), ("node3", True),
                       ("edge4", True), ("node4", True), ("edge5", False),
                       ("node5", False)):
        p = params[name]
        enc[name] = _pool_weights(p["fcx"]["w"], p["fcx"]["b"],
                                  p["fce"]["w"], p["fce"]["b"], pool)

    xx = jnp.zeros((NP, F), jnp.float32).at[:N_NODES, :5].set(x)

    ea = edge_attr
    for r, (ename, nname) in enumerate(
            [("edge1", "node1"), ("edge2", "node2"), ("edge3", "node3"),
             ("edge4", "node4"), ("edge5", "node5")]):
        s0 = jnp.take(xx, src, axis=0)
        s1 = jnp.take(xx, dst, axis=0)
        ea = _fused_pair(s0, s1, ea, enc[ename], N_EDGES, EB)
        adj = jax.ops.segment_sum(ea, src, num_segments=NP)
        xx = _fused_single(xx, adj, enc[nname], NP, NB)

    act = xx
    state = jnp.sum(act[:N_NODES, :6], axis=0)

    fw = [params[n]["w"] for n in ("fc1", "fc2", "fc3", "fc4", "fc5")]
    fb = [params[n]["b"] for n in ("fc1", "fc2", "fc3", "fc4", "fc5")]
    w1 = _pad2(fw[0][:, 6:12].T, F, 128)
    b1 = _pad2((fb[0] + state @ fw[0][:, :6].T)[None, :], 1, 128)
    w2 = _pad2(fw[1].T, 128, 256)
    b2 = _pad2(fb[1][None, :], 1, 256)
    w3 = _pad2(fw[2].T, 256, 128)
    b3 = _pad2(fb[2][None, :], 1, 128)
    w4 = _pad2(fw[3].T, 128, 128)
    b4 = _pad2(fb[3][None, :], 1, 128)
    w5 = _pad2(fw[4].T, 128, 8)
    b5 = _pad2(fb[4][None, :], 1, 8)

    q = _mlp(act, (w1, b1, w2, b2, w3, b3, w4, b4, w5, b5))
    return q[:N_NODES, 0]
